# Initial kernel scaffold; baseline (speedup 1.0000x reference)
import jax
import jax.numpy as jnp
from jax.experimental import pallas as pl


def kernel(x, W):
    raise NotImplementedError("write your pallas kernel here")


# speedup vs baseline: 7.7243x; 7.7243x over previous
import functools

import jax
import jax.numpy as jnp
from jax import lax
from jax.experimental import pallas as pl
from jax.experimental.pallas import tpu as pltpu
from jax.experimental.pallas import tpu_sc as plsc

VOCAB = 100000
EMBED_DIM = 128
BATCH = 4096
SEQ = 200

NC, NS, L = 2, 16, 16
NW = NC * NS
B_TOTAL = BATCH * SEQ
B_PER_W = B_TOTAL // NW
GRP = 128
NGRP = B_PER_W // GRP
GPB = 2
ROWS_PER_STEP = GPB * GRP
NSTEP = NGRP // GPB


@functools.partial(
    pl.kernel,
    out_type=jax.ShapeDtypeStruct((B_TOTAL, EMBED_DIM), jnp.float32),
    mesh=plsc.VectorSubcoreMesh(core_axis_name="c", subcore_axis_name="s"),
    scratch_types=[
        pltpu.VMEM((NGRP, GRP), jnp.int32),
        pltpu.VMEM((ROWS_PER_STEP, EMBED_DIM), jnp.float32),
        pltpu.SemaphoreType.DMA,
    ],
)
def _gather_kernel(x_hbm, w_hbm, out_hbm, idx_v, rows_v, sem):
    wid = lax.axis_index("s") * NC + lax.axis_index("c")
    base = wid * B_PER_W
    pltpu.sync_copy(x_hbm.at[wid], idx_v)

    def step(i, _):
        cps = [
            pltpu.async_copy(
                w_hbm.at[idx_v.at[i * GPB + g]],
                rows_v.at[pl.ds(g * GRP, GRP)],
                sem,
            )
            for g in range(GPB)
        ]
        for cp in cps:
            cp.wait()
        pltpu.sync_copy(
            rows_v, out_hbm.at[pl.ds(base + i * ROWS_PER_STEP, ROWS_PER_STEP)]
        )
        return 0

    lax.fori_loop(0, NSTEP, step, 0)


def kernel(x, W):
    x3 = x.reshape(NW, NGRP, GRP)
    out = _gather_kernel(x3, W)
    return out.reshape(BATCH, SEQ, EMBED_DIM)


# speedup vs baseline: 9.1809x; 1.1886x over previous
import functools

import jax
import jax.numpy as jnp
from jax import lax
from jax.experimental import pallas as pl
from jax.experimental.pallas import tpu as pltpu
from jax.experimental.pallas import tpu_sc as plsc

VOCAB = 100000
EMBED_DIM = 128
BATCH = 4096
SEQ = 200

NC, NS, L = 2, 16, 16
NW = NC * NS
B_TOTAL = BATCH * SEQ
B_PER_W = B_TOTAL // NW
GRP = 128
NGRP = B_PER_W // GRP
GPB = 2
ROWS_PER_STEP = GPB * GRP
NSTEP = NGRP // GPB


NBUF = 2


@functools.partial(
    pl.kernel,
    out_type=jax.ShapeDtypeStruct((B_TOTAL, EMBED_DIM), jnp.float32),
    mesh=plsc.VectorSubcoreMesh(core_axis_name="c", subcore_axis_name="s"),
    scratch_types=[
        pltpu.VMEM((NGRP, GRP), jnp.int32),
        pltpu.VMEM((NBUF, ROWS_PER_STEP, EMBED_DIM), jnp.float32),
        pltpu.SemaphoreType.DMA,
        pltpu.SemaphoreType.DMA,
        pltpu.SemaphoreType.DMA,
    ],
)
def _gather_kernel(x_hbm, w_hbm, out_hbm, idx_v, rows_v, sem_g0, sem_g1, sem_w):
    wid = lax.axis_index("s") * NC + lax.axis_index("c")
    base = wid * B_PER_W
    sems = (sem_g0, sem_g1)
    pltpu.sync_copy(x_hbm.at[wid], idx_v)

    def fire(s, b):
        return [
            pltpu.async_copy(
                w_hbm.at[idx_v.at[s * GPB + g]],
                rows_v.at[b].at[pl.ds(g * GRP, GRP)],
                sems[b],
            )
            for g in range(GPB)
        ]

    prim = [fire(b, b) for b in range(NBUF)]

    def step(i, _):
        for b in range(NBUF):
            s = i * NBUF + b
            for cp in prim[b]:
                cp.wait()
            pltpu.async_copy(
                rows_v.at[b],
                out_hbm.at[pl.ds(base + s * ROWS_PER_STEP, ROWS_PER_STEP)],
                sem_w,
            ).wait()

            @pl.when(s + NBUF < NSTEP)
            def _():
                fire(s + NBUF, b)

        return 0

    lax.fori_loop(0, NSTEP // NBUF, step, 0)


def kernel(x, W):
    x3 = x.reshape(NW, NGRP, GRP)
    out = _gather_kernel(x3, W)
    return out.reshape(BATCH, SEQ, EMBED_DIM)


# speedup vs baseline: 9.1972x; 1.0018x over previous
import functools

import jax
import jax.numpy as jnp
from jax import lax
from jax.experimental import pallas as pl
from jax.experimental.pallas import tpu as pltpu
from jax.experimental.pallas import tpu_sc as plsc

VOCAB = 100000
EMBED_DIM = 128
BATCH = 4096
SEQ = 200

NC, NS, L = 2, 16, 16
NW = NC * NS
B_TOTAL = BATCH * SEQ
B_PER_W = B_TOTAL // NW
GRP = 128
NGRP = B_PER_W // GRP
GPB = 1
ROWS_PER_STEP = GPB * GRP
NSTEP = NGRP // GPB


NBUF = 4


@functools.partial(
    pl.kernel,
    out_type=jax.ShapeDtypeStruct((B_TOTAL, EMBED_DIM), jnp.float32),
    mesh=plsc.VectorSubcoreMesh(core_axis_name="c", subcore_axis_name="s"),
    scratch_types=[
        pltpu.VMEM((NGRP, GRP), jnp.int32),
        pltpu.VMEM((NBUF, ROWS_PER_STEP, EMBED_DIM), jnp.float32),
        pltpu.SemaphoreType.DMA,
        pltpu.SemaphoreType.DMA,
        pltpu.SemaphoreType.DMA,
        pltpu.SemaphoreType.DMA,
        pltpu.SemaphoreType.DMA,
    ],
)
def _gather_kernel(x_hbm, w_hbm, out_hbm, idx_v, rows_v,
                   sem_g0, sem_g1, sem_g2, sem_g3, sem_w):
    wid = lax.axis_index("s") * NC + lax.axis_index("c")
    base = wid * B_PER_W
    sems = (sem_g0, sem_g1, sem_g2, sem_g3)
    pltpu.sync_copy(x_hbm.at[wid], idx_v)

    def fire(s, b):
        return [
            pltpu.async_copy(
                w_hbm.at[idx_v.at[s * GPB + g]],
                rows_v.at[b].at[pl.ds(g * GRP, GRP)],
                sems[b],
            )
            for g in range(GPB)
        ]

    prim = [fire(b, b) for b in range(NBUF)]

    def step(i, _):
        for b in range(NBUF):
            s = i * NBUF + b
            for cp in prim[b]:
                cp.wait()
            pltpu.async_copy(
                rows_v.at[b],
                out_hbm.at[pl.ds(base + s * ROWS_PER_STEP, ROWS_PER_STEP)],
                sem_w,
            ).wait()

            @pl.when(s + NBUF < NSTEP)
            def _():
                fire(s + NBUF, b)

        return 0

    lax.fori_loop(0, NSTEP // NBUF, step, 0)


def kernel(x, W):
    x3 = x.reshape(NW, NGRP, GRP)
    out = _gather_kernel(x3, W)
    return out.reshape(BATCH, SEQ, EMBED_DIM)
